# baseline (speedup 1.0000x reference)
import functools

import jax
import jax.numpy as jnp
from jax import lax
from jax.experimental import pallas as pl
from jax.experimental.pallas import tpu as pltpu
from jax.experimental.pallas import tpu_sc as plsc

N = 10000
E = 320000
D = 128
H = 128

NC = 2
NS = 16
L = 16
NW = NC * NS

CH = 128
KCH = 88
GRP = 8
EP = NW * KCH * CH
PADE = EP - (E + N)
NPD = 10240
RPD = NPD // NS
NB = 10112
NPASS = 5
AR = 2176
RR = 2048
RPA = AR // NS
YR = 10000
PK = 16384
EPT = KCH * CH
ARENA = 12176
AEXP = 12160

_mesh = plsc.VectorSubcoreMesh(core_axis_name="c", subcore_axis_name="s")


@functools.partial(
    pl.kernel,
    mesh=_mesh,
    out_type=jax.ShapeDtypeStruct((NC * NPD,), jnp.float32),
    scratch_types=[
        pltpu.VMEM((KCH, CH), jnp.int32),
        pltpu.VMEM((CH,), jnp.float32),
        pltpu.VMEM_SHARED((NPD,), jnp.float32),
        pltpu.SemaphoreType.DMA,
    ],
)
def _deg_kernel(cols_hbm, zeros1_hbm, out_hbm, colv, ones_v, acc, sem):
    c = lax.axis_index("c")
    s = lax.axis_index("s")
    wid = s * NC + c
    pltpu.sync_copy(zeros1_hbm.at[pl.ds(s * RPD, RPD)],
                    acc.at[pl.ds(s * RPD, RPD)])
    for i in range(CH // L):
        ones_v[pl.ds(i * L, L)] = jnp.ones((L,), jnp.float32)
    pltpu.sync_copy(cols_hbm.at[pl.ds(wid * KCH, KCH)], colv)
    plsc.subcore_barrier()

    def body(j, _):
        pltpu.sync_copy(ones_v, acc.at[colv.at[j]], add=True)
        return 0

    lax.fori_loop(0, KCH, body, 0)
    plsc.subcore_barrier()
    pltpu.sync_copy(acc.at[pl.ds(s * RPD, RPD)],
                    out_hbm.at[pl.ds(c * NPD + s * RPD, RPD)])


def _make_agg(build):
    outs = jax.ShapeDtypeStruct((NC, NPASS, AR, H), jnp.float32)
    if build:
        outs = [outs,
                jax.ShapeDtypeStruct((NW * AEXP,), jnp.int32),
                jax.ShapeDtypeStruct((NW * CH,), jnp.int32)]

    @functools.partial(
        pl.kernel,
        mesh=_mesh,
        compiler_params=pltpu.CompilerParams(needs_layout_passes=False),
        out_type=outs,
        scratch_types=[
            pltpu.VMEM((GRP, CH), jnp.int32),
            pltpu.VMEM((ARENA,), jnp.int32),
            pltpu.VMEM((CH,), jnp.int32),
            pltpu.VMEM((CH,), jnp.int32),
            pltpu.VMEM((CH,), jnp.int32),
            pltpu.VMEM((CH, H), jnp.float32),
            pltpu.VMEM_SHARED((YR, H), jnp.float32),
            pltpu.VMEM_SHARED((AR, H), jnp.float32),
            pltpu.SemaphoreType.DMA,
        ],
    )
    def agg(*refs):
        if build:
            (y_hbm, pk_hbm, zerosa_hbm, out_hbm, arena_hbm, cnts_hbm,
             ring, arena, rowc, colc, cntb, gbuf, ysp, acc, sem) = refs
        else:
            (y_hbm, arena_in_hbm, cnts_in_hbm, zerosa_hbm, out_hbm,
             ring, arena, rowc, colc, cntb, gbuf, ysp, acc, sem) = refs
        c = lax.axis_index("c")
        s = lax.axis_index("s")
        wid = s * NC + c
        iota = lax.iota(jnp.int32, L)

        @pl.when(s < NS - 1)
        def _():
            pltpu.sync_copy(y_hbm.at[pl.ds(s * 632, 632)],
                            ysp.at[pl.ds(s * 632, 632)])

        @pl.when(s == NS - 1)
        def _():
            pltpu.sync_copy(
                y_hbm.at[pl.ds((NS - 1) * 632, YR - (NS - 1) * 632)],
                ysp.at[pl.ds((NS - 1) * 632, YR - (NS - 1) * 632)])

        pltpu.sync_copy(zerosa_hbm.at[pl.ds(s * RPA, RPA)],
                        acc.at[pl.ds(s * RPA, RPA)])

        nchunks = []
        if build:
            off = jnp.int32(0)
            starts = []
            for p in range(NPASS):
                starts.append(off)
                pbkt = jnp.int32(p)

                def grp_body(g, off, pbkt=pbkt):
                    pltpu.sync_copy(
                        pk_hbm.at[pl.ds(wid * KCH + g * GRP, GRP)], ring)
                    for j in range(GRP):
                        for i in range(CH // L):
                            v = ring[j, pl.ds(i * L, L)]
                            col = jnp.bitwise_and(v, jnp.int32(PK - 1))
                            m = jnp.right_shift(col, jnp.int32(11)) == pbkt
                            mi = jnp.where(m, jnp.int32(1), jnp.int32(0))
                            pos = jnp.where(
                                m, off + plsc.cumsum(mi) - 1,
                                jnp.int32(ARENA - L) + iota)
                            plsc.store_scatter(arena, [pos], v)
                            off = off + jnp.sum(mi)
                    return off

                off = lax.fori_loop(0, KCH // GRP, grp_body, off)
                dummy = jnp.full((L,), p * RR + RR, jnp.int32) + iota
                for k in range(CH // L):
                    plsc.store_scatter(
                        arena, [off + jnp.int32(k * L) + iota], dummy)
                cnt = off - starts[p]
                nchunks.append(jnp.right_shift(cnt + jnp.int32(CH - 1),
                                               jnp.int32(7)))
                off = starts[p] + nchunks[p] * CH
            cvec = jnp.zeros((L,), jnp.int32)
            for p in range(NPASS):
                cvec = cvec + jnp.where(iota == p, nchunks[p], jnp.int32(0))
            for k in range(CH // L):
                cntb[pl.ds(k * L, L)] = cvec
            pltpu.sync_copy(arena.at[pl.ds(0, AEXP)],
                            arena_hbm.at[pl.ds(wid * AEXP, AEXP)])
            pltpu.sync_copy(cntb, cnts_hbm.at[pl.ds(wid * CH, CH)])
        else:
            pltpu.sync_copy(arena_in_hbm.at[pl.ds(wid * AEXP, AEXP)],
                            arena.at[pl.ds(0, AEXP)])
            pltpu.sync_copy(cnts_in_hbm.at[pl.ds(wid * CH, CH)], cntb)
            cvec = cntb[pl.ds(0, L)]
            for p in range(NPASS):
                nchunks.append(jnp.sum(jnp.where(iota == p, cvec,
                                                 jnp.int32(0))))

        plsc.subcore_barrier()

        start = jnp.int32(0)
        for p in range(NPASS):
            base_p = jnp.int32(p * RR)

            def chunk_body(ch, _, start=start, base_p=base_p):
                base = start + ch * CH
                for i in range(CH // L):
                    v = plsc.load_gather(
                        arena, [base + jnp.int32(i * L) + iota])
                    rowc[pl.ds(i * L, L)] = jnp.right_shift(v, jnp.int32(14))
                    colc[pl.ds(i * L, L)] = (
                        jnp.bitwise_and(v, jnp.int32(PK - 1)) - base_p)
                pltpu.async_copy(ysp.at[rowc], gbuf, sem).wait()
                pltpu.sync_copy(gbuf, acc.at[colc], add=True)
                return 0

            lax.fori_loop(0, nchunks[p], chunk_body, 0)
            start = start + nchunks[p] * CH
            plsc.subcore_barrier()
            pltpu.sync_copy(acc.at[pl.ds(s * RPA, RPA)],
                            out_hbm.at[c, p, pl.ds(s * RPA, RPA)])
            if p < NPASS - 1:
                pltpu.sync_copy(zerosa_hbm.at[pl.ds(s * RPA, RPA)],
                                acc.at[pl.ds(s * RPA, RPA)])
            plsc.subcore_barrier()

    return agg


_agg_build = _make_agg(True)
_agg_reuse = _make_agg(False)


def _mm_scale_body(x_ref, w_ref, d0_ref, d1_ref, y_ref, dinv_ref):
    dinv = lax.rsqrt(d0_ref[...] + d1_ref[...])
    y_ref[...] = jnp.dot(x_ref[...], w_ref[...],
                         preferred_element_type=jnp.float32) * dinv
    dinv_ref[...] = dinv


_mm_scale = pl.pallas_call(
    _mm_scale_body,
    out_shape=[jax.ShapeDtypeStruct((N, H), jnp.float32),
               jax.ShapeDtypeStruct((N, 1), jnp.float32)],
)


def _stitch(p_ref):
    q0 = p_ref[0]
    q1 = p_ref[1]
    parts = [q0[p, :RR, :] + q1[p, :RR, :] for p in range(NPASS - 1)]
    parts.append(q0[NPASS - 1, :N - (NPASS - 1) * RR, :]
                 + q1[NPASS - 1, :N - (NPASS - 1) * RR, :])
    return jnp.concatenate(parts, axis=0)


def _mid_body(p_ref, dinv_ref, b_ref, w_ref, y_ref):
    dinv = dinv_ref[...]
    h = jnp.maximum(_stitch(p_ref) * dinv + b_ref[...], 0.0)
    y_ref[...] = jnp.dot(h, w_ref[...],
                         preferred_element_type=jnp.float32) * dinv


_mid = pl.pallas_call(
    _mid_body,
    out_shape=jax.ShapeDtypeStruct((N, H), jnp.float32),
)


def _final_body(p_ref, dinv_ref, b_ref, out_ref):
    out_ref[...] = jnp.maximum(
        _stitch(p_ref) * dinv_ref[...] + b_ref[...], 0.0)


_final = pl.pallas_call(
    _final_body,
    out_shape=jax.ShapeDtypeStruct((N, H), jnp.float32),
)


def kernel(x, edge_index, W1, b1, W2, b2):
    loop = jnp.arange(N, dtype=jnp.int32)
    rows_all = jnp.concatenate(
        [edge_index[0], loop, jnp.zeros((PADE,), jnp.int32)])
    cols_all = jnp.concatenate(
        [edge_index[1], loop, N + jnp.arange(PADE, dtype=jnp.int32) % (NB - N)]
    )
    cols_deg = cols_all.reshape(NW * KCH, CH)
    packed = (rows_all * PK + cols_all).reshape(NW * KCH, CH)
    zeros1 = jnp.zeros((NPD,), jnp.float32)
    zerosa = jnp.zeros((AR, H), jnp.float32)

    degp = _deg_kernel(cols_deg, zeros1)
    d0 = degp[:N].reshape(N, 1)
    d1 = degp[NPD:NPD + N].reshape(N, 1)

    y1, dinv = _mm_scale(x, W1, d0, d1)
    p1, arena, cnts = _agg_build(y1, packed, zerosa)
    y2 = _mid(p1, dinv, b1.reshape(1, H), W2)
    p2 = _agg_reuse(y2, arena, cnts, zerosa)
    return _final(p2, dinv, b2.reshape(1, H))


# speedup vs baseline: 1.2217x; 1.2217x over previous
import functools

import jax
import jax.numpy as jnp
from jax import lax
from jax.experimental import pallas as pl
from jax.experimental.pallas import tpu as pltpu
from jax.experimental.pallas import tpu_sc as plsc

N = 10000
E = 320000
D = 128
H = 128

NC = 2
NS = 16
L = 16
NW = NC * NS

CH = 128
KCH = 88
GRP = 8
CH2 = 64
EP = NW * KCH * CH
PADE = EP - (E + N)
NPD = 10240
RPD = NPD // NS
NB = 10112
NPASS = 5
AR = 2176
RR = 2048
RPA = AR // NS
YR = 10000
PK = 16384
EPT = KCH * CH
ARENA = 12176
AEXP = 12160

_mesh = plsc.VectorSubcoreMesh(core_axis_name="c", subcore_axis_name="s")


@functools.partial(
    pl.kernel,
    mesh=_mesh,
    out_type=jax.ShapeDtypeStruct((NC * NPD,), jnp.float32),
    scratch_types=[
        pltpu.VMEM((KCH, CH), jnp.int32),
        pltpu.VMEM((CH,), jnp.float32),
        pltpu.VMEM_SHARED((NPD,), jnp.float32),
        pltpu.SemaphoreType.DMA,
    ],
)
def _deg_kernel(cols_hbm, zeros1_hbm, out_hbm, colv, ones_v, acc, sem):
    c = lax.axis_index("c")
    s = lax.axis_index("s")
    wid = s * NC + c
    pltpu.sync_copy(zeros1_hbm.at[pl.ds(s * RPD, RPD)],
                    acc.at[pl.ds(s * RPD, RPD)])
    for i in range(CH // L):
        ones_v[pl.ds(i * L, L)] = jnp.ones((L,), jnp.float32)
    pltpu.sync_copy(cols_hbm.at[pl.ds(wid * KCH, KCH)], colv)
    plsc.subcore_barrier()

    def body(j, _):
        pltpu.sync_copy(ones_v, acc.at[colv.at[j]], add=True)
        return 0

    lax.fori_loop(0, KCH, body, 0)
    plsc.subcore_barrier()
    pltpu.sync_copy(acc.at[pl.ds(s * RPD, RPD)],
                    out_hbm.at[pl.ds(c * NPD + s * RPD, RPD)])


def _make_agg(build):
    outs = jax.ShapeDtypeStruct((NC, NPASS, AR, H), jnp.float32)
    if build:
        outs = [outs,
                jax.ShapeDtypeStruct((NW * AEXP,), jnp.int32),
                jax.ShapeDtypeStruct((NW * CH,), jnp.int32)]

    @functools.partial(
        pl.kernel,
        mesh=_mesh,
        compiler_params=pltpu.CompilerParams(needs_layout_passes=False),
        out_type=outs,
        scratch_types=[
            pltpu.VMEM((GRP, CH), jnp.int32),
            pltpu.VMEM((ARENA,), jnp.int32),
            pltpu.VMEM((CH2,), jnp.int32),
            pltpu.VMEM((CH2,), jnp.int32),
            pltpu.VMEM((CH2,), jnp.int32),
            pltpu.VMEM((CH2,), jnp.int32),
            pltpu.VMEM((CH,), jnp.int32),
            pltpu.VMEM((CH2, H), jnp.float32),
            pltpu.VMEM((CH2, H), jnp.float32),
            pltpu.VMEM_SHARED((YR, H), jnp.float32),
            pltpu.VMEM_SHARED((AR, H), jnp.float32),
            pltpu.SemaphoreType.DMA,
            pltpu.SemaphoreType.DMA,
        ],
    )
    def agg(*refs):
        if build:
            (y_hbm, pk_hbm, zerosa_hbm, out_hbm, arena_hbm, cnts_hbm,
             ring, arena, rowc0, colc0, rowc1, colc1, cntb, gbuf0, gbuf1,
             ysp, acc, sem0, sem1) = refs
        else:
            (y_hbm, arena_in_hbm, cnts_in_hbm, zerosa_hbm, out_hbm,
             ring, arena, rowc0, colc0, rowc1, colc1, cntb, gbuf0, gbuf1,
             ysp, acc, sem0, sem1) = refs
        c = lax.axis_index("c")
        s = lax.axis_index("s")
        wid = s * NC + c
        iota = lax.iota(jnp.int32, L)

        @pl.when(s < NS - 1)
        def _():
            pltpu.sync_copy(y_hbm.at[pl.ds(s * 632, 632)],
                            ysp.at[pl.ds(s * 632, 632)])

        @pl.when(s == NS - 1)
        def _():
            pltpu.sync_copy(
                y_hbm.at[pl.ds((NS - 1) * 632, YR - (NS - 1) * 632)],
                ysp.at[pl.ds((NS - 1) * 632, YR - (NS - 1) * 632)])

        pltpu.sync_copy(zerosa_hbm.at[pl.ds(s * RPA, RPA)],
                        acc.at[pl.ds(s * RPA, RPA)])

        nchunks = []
        if build:
            off = jnp.int32(0)
            starts = []
            for p in range(NPASS):
                starts.append(off)
                pbkt = jnp.int32(p)

                def grp_body(g, off, pbkt=pbkt):
                    pltpu.sync_copy(
                        pk_hbm.at[pl.ds(wid * KCH + g * GRP, GRP)], ring)
                    for j in range(GRP):
                        for i in range(CH // L):
                            v = ring[j, pl.ds(i * L, L)]
                            col = jnp.bitwise_and(v, jnp.int32(PK - 1))
                            m = jnp.right_shift(col, jnp.int32(11)) == pbkt
                            mi = jnp.where(m, jnp.int32(1), jnp.int32(0))
                            pos = jnp.where(
                                m, off + plsc.cumsum(mi) - 1,
                                jnp.int32(ARENA - L) + iota)
                            plsc.store_scatter(arena, [pos], v)
                            off = off + jnp.sum(mi)
                    return off

                off = lax.fori_loop(0, KCH // GRP, grp_body, off)
                dummy = jnp.full((L,), p * RR + RR, jnp.int32) + iota
                for k in range(CH // L):
                    plsc.store_scatter(
                        arena, [off + jnp.int32(k * L) + iota], dummy)
                cnt = off - starts[p]
                nchunks.append(jnp.right_shift(cnt + jnp.int32(CH - 1),
                                               jnp.int32(7)))
                off = starts[p] + nchunks[p] * CH
            cvec = jnp.zeros((L,), jnp.int32)
            for p in range(NPASS):
                cvec = cvec + jnp.where(iota == p, nchunks[p], jnp.int32(0))
            for k in range(CH // L):
                cntb[pl.ds(k * L, L)] = cvec
            pltpu.sync_copy(arena.at[pl.ds(0, AEXP)],
                            arena_hbm.at[pl.ds(wid * AEXP, AEXP)])
            pltpu.sync_copy(cntb, cnts_hbm.at[pl.ds(wid * CH, CH)])
        else:
            pltpu.sync_copy(arena_in_hbm.at[pl.ds(wid * AEXP, AEXP)],
                            arena.at[pl.ds(0, AEXP)])
            pltpu.sync_copy(cnts_in_hbm.at[pl.ds(wid * CH, CH)], cntb)
            cvec = cntb[pl.ds(0, L)]
            for p in range(NPASS):
                nchunks.append(jnp.sum(jnp.where(iota == p, cvec,
                                                 jnp.int32(0))))

        plsc.subcore_barrier()

        def unpack(base, rowb, colb, base_p):
            for i in range(CH2 // L):
                v = plsc.load_gather(arena, [base + jnp.int32(i * L) + iota])
                rowb[pl.ds(i * L, L)] = jnp.right_shift(v, jnp.int32(14))
                colb[pl.ds(i * L, L)] = (
                    jnp.bitwise_and(v, jnp.int32(PK - 1)) - base_p)

        start = jnp.int32(0)
        for p in range(NPASS):
            base_p = jnp.int32(p * RR)
            npair = nchunks[p]

            @pl.when(npair > 0)
            def _(start=start, base_p=base_p):
                unpack(start, rowc0, colc0, base_p)
                pltpu.async_copy(ysp.at[rowc0], gbuf0, sem0)

            def pair_body(kk, _, start=start, base_p=base_p, npair=npair):
                base = start + kk * CH
                unpack(base + CH2, rowc1, colc1, base_p)
                pltpu.async_copy(ysp.at[rowc1], gbuf1, sem1)
                pltpu.make_async_copy(ysp.at[rowc0], gbuf0, sem0).wait()
                pltpu.sync_copy(gbuf0, acc.at[colc0], add=True)

                @pl.when(kk < npair - 1)
                def _():
                    unpack(base + CH, rowc0, colc0, base_p)
                    pltpu.async_copy(ysp.at[rowc0], gbuf0, sem0)

                pltpu.make_async_copy(ysp.at[rowc1], gbuf1, sem1).wait()
                pltpu.sync_copy(gbuf1, acc.at[colc1], add=True)
                return 0

            lax.fori_loop(0, npair, pair_body, 0)
            start = start + nchunks[p] * CH
            plsc.subcore_barrier()
            pltpu.sync_copy(acc.at[pl.ds(s * RPA, RPA)],
                            out_hbm.at[c, p, pl.ds(s * RPA, RPA)])
            if p < NPASS - 1:
                pltpu.sync_copy(zerosa_hbm.at[pl.ds(s * RPA, RPA)],
                                acc.at[pl.ds(s * RPA, RPA)])
            plsc.subcore_barrier()

    return agg


_agg_build = _make_agg(True)
_agg_reuse = _make_agg(False)


def _mm_scale_body(x_ref, w_ref, d0_ref, d1_ref, y_ref, dinv_ref):
    dinv = lax.rsqrt(d0_ref[...] + d1_ref[...])
    y_ref[...] = jnp.dot(x_ref[...], w_ref[...],
                         preferred_element_type=jnp.float32) * dinv
    dinv_ref[...] = dinv


_mm_scale = pl.pallas_call(
    _mm_scale_body,
    out_shape=[jax.ShapeDtypeStruct((N, H), jnp.float32),
               jax.ShapeDtypeStruct((N, 1), jnp.float32)],
)


def _stitch(p_ref):
    q0 = p_ref[0]
    q1 = p_ref[1]
    parts = [q0[p, :RR, :] + q1[p, :RR, :] for p in range(NPASS - 1)]
    parts.append(q0[NPASS - 1, :N - (NPASS - 1) * RR, :]
                 + q1[NPASS - 1, :N - (NPASS - 1) * RR, :])
    return jnp.concatenate(parts, axis=0)


def _mid_body(p_ref, dinv_ref, b_ref, w_ref, y_ref):
    dinv = dinv_ref[...]
    h = jnp.maximum(_stitch(p_ref) * dinv + b_ref[...], 0.0)
    y_ref[...] = jnp.dot(h, w_ref[...],
                         preferred_element_type=jnp.float32) * dinv


_mid = pl.pallas_call(
    _mid_body,
    out_shape=jax.ShapeDtypeStruct((N, H), jnp.float32),
)


def _final_body(p_ref, dinv_ref, b_ref, out_ref):
    out_ref[...] = jnp.maximum(
        _stitch(p_ref) * dinv_ref[...] + b_ref[...], 0.0)


_final = pl.pallas_call(
    _final_body,
    out_shape=jax.ShapeDtypeStruct((N, H), jnp.float32),
)


def kernel(x, edge_index, W1, b1, W2, b2):
    loop = jnp.arange(N, dtype=jnp.int32)
    rows_all = jnp.concatenate(
        [edge_index[0], loop, jnp.zeros((PADE,), jnp.int32)])
    cols_all = jnp.concatenate(
        [edge_index[1], loop, N + jnp.arange(PADE, dtype=jnp.int32) % (NB - N)]
    )
    cols_deg = cols_all.reshape(NW * KCH, CH)
    packed = (rows_all * PK + cols_all).reshape(NW * KCH, CH)
    zeros1 = jnp.zeros((NPD,), jnp.float32)
    zerosa = jnp.zeros((AR, H), jnp.float32)

    degp = _deg_kernel(cols_deg, zeros1)
    d0 = degp[:N].reshape(N, 1)
    d1 = degp[NPD:NPD + N].reshape(N, 1)

    y1, dinv = _mm_scale(x, W1, d0, d1)
    p1, arena, cnts = _agg_build(y1, packed, zerosa)
    y2 = _mid(p1, dinv, b1.reshape(1, H), W2)
    p2 = _agg_reuse(y2, arena, cnts, zerosa)
    return _final(p2, dinv, b2.reshape(1, H))
